# baseline (speedup 1.0000x reference)
import functools

import jax
import jax.numpy as jnp
from jax import lax
from jax.experimental import pallas as pl
from jax.experimental.pallas import tpu as pltpu
from jax.experimental.pallas import tpu_sc as plsc

SD, LD, VD, RD, AD = 16, 3, 6, 10, 12
RA = RD * AD
ROWP = 128
L12 = SD * VD * ROWP
TBLP = LD * L12
TSH = VD * ROWP
SLC = LD * TSH
NC, NS, LN = 2, 16, 16
NW = NC * NS
CHUNK = 3200
ATT_PER_DATUM = 0.99999


def _hist_body(nchunks, niter, src_h, lab_h, var_h, ref_h, alt_h,
               idx_h, hist_h,
               b00, b01, b02, b03, b04, b10, b11, b12, b13, b14,
               ix0, ix1, hist_v, sem_in, sem_out):
  wid = lax.axis_index("s") * NC + lax.axis_index("c")
  ins = (src_h, lab_h, var_h, ref_h, alt_h)
  bufs = ((b00, b01, b02, b03, b04), (b10, b11, b12, b13, b14))
  ixv = (ix0, ix1)

  def zero(j, c):
    o = j * (4 * LN)
    for u in range(4):
      hist_v[pl.ds(o + u * LN, LN)] = jnp.zeros((LN,), jnp.float32)
    return c
  lax.fori_loop(0, TBLP // (4 * LN), zero, 0)

  ones = jnp.full((LN,), 1.0, jnp.float32)

  def issue(bsel, cid):
    base = cid * CHUNK
    for h, v in zip(ins, bufs[bsel]):
      pltpu.async_copy(h.at[pl.ds(base, CHUNK)], v, sem_in)

  def drain_in(bsel):
    for h, v in zip(ins, bufs[bsel]):
      pltpu.make_async_copy(h.at[pl.ds(0, CHUNK)], v, sem_in).wait()

  def drain_out(bsel):
    pltpu.make_async_copy(ixv[bsel], idx_h.at[pl.ds(0, CHUNK)], sem_out).wait()

  issue(0, wid)

  def loop_g(g, c):
    for b in range(2):
      cid = wid + (2 * g + b) * NW

      @pl.when(cid < nchunks)
      def _(b=b, cid=cid):
        drain_in(b)

        @pl.when(cid + NW < nchunks)
        def _():
          issue(1 - b, cid + NW)

        @pl.when(g > 0)
        def _():
          drain_out(b)

        sv_v, lv_v, vv_v, rv_v, av_v = bufs[b]
        ixb = ixv[b]

        @plsc.parallel_loop(0, CHUNK, step=LN, unroll=8)
        def _(o):
          s = sv_v[pl.ds(o, LN)]
          l = lv_v[pl.ds(o, LN)]
          v = vv_v[pl.ds(o, LN)]
          r = rv_v[pl.ds(o, LN)]
          a = av_v[pl.ds(o, LN)]
          t = ((l * SD + s) * VD + v) * ROWP + r * AD + a
          ixb[pl.ds(o, LN)] = t | (s << 16)
          plsc.addupdate_scatter(hist_v, [t], ones)
        pltpu.async_copy(ixb, idx_h.at[pl.ds(cid * CHUNK, CHUNK)], sem_out)
    return c
  lax.fori_loop(0, niter // 2, loop_g, 0)
  drain_out(0)
  drain_out(1)

  pltpu.sync_copy(hist_v, hist_h.at[wid])


def _bal_gather_body(att, nchunks, niter, hist_h, c0_h, w0_h, sw0_h, idx_h,
                     bw_h, bsw_h,
                     acc_v, stg0, stg1, c0_v, w0_v, wsl_v, wtab_v, sw16_v,
                     cs_v, ex_v, ix0, ix1, bw0, bw1, bsw0, bsw1,
                     spw_s, spx_s, sem_cfg, sem_tbl, sem_in, sem_out):
  sid = lax.axis_index("s")
  wid = sid * NC + lax.axis_index("c")
  one_m = 1.0 - att
  ixv = (ix0, ix1)
  bwv = (bw0, bw1)
  bswv = (bsw0, bsw1)
  stg = (stg0, stg1)

  pltpu.async_copy(idx_h.at[pl.ds(wid * CHUNK, CHUNK)], ix0, sem_in)

  sbase = sid * TSH
  cfg = []
  for l in range(LD):
    cfg.append(pltpu.async_copy(
        c0_h.at[pl.ds(l * L12 + sbase, TSH)],
        c0_v.at[pl.ds(l * TSH, TSH)], sem_cfg))
    cfg.append(pltpu.async_copy(
        w0_h.at[pl.ds(l * L12 + sbase, TSH)],
        w0_v.at[pl.ds(l * TSH, TSH)], sem_cfg))
  cfg.append(pltpu.async_copy(sw0_h, sw16_v, sem_cfg))

  def issue_w(bsel, w):
    for l in range(LD):
      pltpu.async_copy(
          hist_h.at[w, pl.ds(l * L12 + sbase, TSH)],
          stg[bsel].at[pl.ds(l * TSH, TSH)], sem_tbl)

  def drain_w(bsel):
    for l in range(LD):
      pltpu.make_async_copy(
          hist_h.at[0, pl.ds(l * L12 + sbase, TSH)],
          stg[bsel].at[pl.ds(l * TSH, TSH)], sem_tbl).wait()

  issue_w(0, 0)

  def zacc(j, c):
    o = j * (4 * LN)
    for u in range(4):
      acc_v[pl.ds(o + u * LN, LN)] = jnp.zeros((LN,), jnp.float32)
    return c
  lax.fori_loop(0, SLC // (4 * LN), zacc, 0)

  def wacc(g, c):
    for b in range(2):
      w = 2 * g + b
      drain_w(b)

      @pl.when(w < NW - 1)
      def _(b=b, w=w):
        issue_w(1 - b, w + 1)

      @plsc.parallel_loop(0, SLC, step=LN, unroll=8)
      def _(o, b=b):
        acc_v[pl.ds(o, LN)] = acc_v[pl.ds(o, LN)] + stg[b][pl.ds(o, LN)]
    return c
  lax.fori_loop(0, NW // 2, wacc, 0)

  for d in cfg:
    d.wait()

  def addc(j, c):
    o = j * LN
    acc_v[pl.ds(o, LN)] = acc_v[pl.ds(o, LN)] + c0_v[pl.ds(o, LN)]
    return c
  lax.fori_loop(0, SLC // LN, addc, 0)

  csum = jnp.zeros((LN,), jnp.float32)
  for v in range(VD):
    a_acc = jnp.zeros((LN,), jnp.float32)
    v_acc = jnp.zeros((LN,), jnp.float32)
    u_acc = jnp.zeros((LN,), jnp.float32)
    for j in range(ROWP // LN):
      o = v * ROWP + j * LN
      av = acc_v[pl.ds(o, LN)]
      vv = acc_v[pl.ds(TSH + o, LN)]
      uv = acc_v[pl.ds(2 * TSH + o, LN)]
      a_acc += av
      v_acc += vv
      u_acc += uv
      ratios = (av + 0.01) / (vv + 0.01)
      inv = (vv + 0.01) / (av + 0.01)
      w_art = jnp.clip((1.0 + inv) / 2.0, 0.01, 100.0)
      w_var = jnp.clip((1.0 + ratios) / 2.0, 0.01, 100.0)
      wsl_v[pl.ds(o, LN)] = att * w0_v[pl.ds(o, LN)] + one_m * w_art
      wsl_v[pl.ds(TSH + o, LN)] = (
          att * w0_v[pl.ds(TSH + o, LN)] + one_m * w_var)
    csum += a_acc + v_acc + u_acc
    sa = jnp.broadcast_to(jnp.sum(a_acc), (LN,))
    sv = jnp.broadcast_to(jnp.sum(v_acc), (LN,))
    su = jnp.broadcast_to(jnp.sum(u_acc), (LN,))
    uw = jnp.clip((sa + sv) / su, 0.0, 1.0)
    for j in range(ROWP // LN):
      o = 2 * TSH + v * ROWP + j * LN
      wsl_v[pl.ds(o, LN)] = att * w0_v[pl.ds(o, LN)] + one_m * uw

  cs_v[pl.ds(0, LN)] = jnp.broadcast_to(jnp.sum(csum), (LN,))
  pltpu.sync_copy(cs_v, spx_s.at[pl.ds(sid * LN, LN)])
  for l in range(LD):
    pltpu.sync_copy(wsl_v.at[pl.ds(l * TSH, TSH)],
                    spw_s.at[pl.ds(l * L12 + sbase, TSH)])
  plsc.subcore_barrier()
  pltpu.sync_copy(spw_s, wtab_v)
  pltpu.sync_copy(spx_s, ex_v)

  iota16 = lax.iota(jnp.int32, LN)
  csrow = plsc.load_gather(ex_v, [iota16 * LN])
  totv = jnp.broadcast_to(jnp.sum(csrow), (LN,))
  new_sw = totv / csrow / SD
  sw16_v[pl.ds(0, LN)] = att * sw16_v[pl.ds(0, LN)] + one_m * new_sw

  def drain_gout(bsel):
    pltpu.make_async_copy(bwv[bsel], bw_h.at[pl.ds(0, CHUNK)], sem_out).wait()
    pltpu.make_async_copy(bswv[bsel], bsw_h.at[pl.ds(0, CHUNK)], sem_out).wait()

  def loop_g(g, c):
    for b in range(2):
      cid = wid + (2 * g + b) * NW

      @pl.when(cid < nchunks)
      def _(b=b, cid=cid):
        pltpu.make_async_copy(
            idx_h.at[pl.ds(0, CHUNK)], ixv[b], sem_in).wait()

        @pl.when(cid + NW < nchunks)
        def _():
          pltpu.async_copy(
              idx_h.at[pl.ds((cid + NW) * CHUNK, CHUNK)], ixv[1 - b], sem_in)

        @pl.when(g > 0)
        def _():
          drain_gout(b)

        ixb, bwb, bswb = ixv[b], bwv[b], bswv[b]

        @plsc.parallel_loop(0, CHUNK, step=LN, unroll=8)
        def _(o):
          t = ixb[pl.ds(o, LN)]
          tl = t & 0xFFFF
          s = t >> 16
          bwb[pl.ds(o, LN)] = plsc.load_gather(wtab_v, [tl])
          bswb[pl.ds(o, LN)] = plsc.load_gather(sw16_v, [s])
        base = cid * CHUNK
        pltpu.async_copy(bwb, bw_h.at[pl.ds(base, CHUNK)], sem_out)
        pltpu.async_copy(bswb, bsw_h.at[pl.ds(base, CHUNK)], sem_out)
    return c
  lax.fori_loop(0, niter // 2, loop_g, 0)
  drain_gout(0)
  drain_gout(1)


def kernel(counts_slvra, weights_slvra, source_weights_s,
           sources, labels, var_types, ref_bins, alt_bins):
  b = sources.shape[0]
  assert b % CHUNK == 0
  nchunks = b // CHUNK
  assert nchunks >= 2 * NW
  niter = -(-nchunks // NW)
  niter += niter % 2
  att = float(ATT_PER_DATUM ** b)
  mesh = plsc.VectorSubcoreMesh(core_axis_name="c", subcore_axis_name="s")

  hist_call = pl.kernel(
      functools.partial(_hist_body, nchunks, niter),
      out_type=[
          jax.ShapeDtypeStruct((b,), jnp.int32),
          jax.ShapeDtypeStruct((NW, TBLP), jnp.float32),
      ],
      mesh=mesh,
      compiler_params=pltpu.CompilerParams(needs_layout_passes=False),
      scratch_types=(
          [pltpu.VMEM((CHUNK,), jnp.int32) for _ in range(12)]
          + [pltpu.VMEM((TBLP,), jnp.float32)]
          + [pltpu.SemaphoreType.DMA, pltpu.SemaphoreType.DMA]
      ),
  )
  idx, hist32 = hist_call(sources, labels, var_types, ref_bins, alt_bins)

  c0p = jnp.pad(
      jnp.transpose(counts_slvra, (1, 0, 2, 3, 4)).reshape(LD, SD * VD, RA),
      ((0, 0), (0, 0), (0, ROWP - RA))).reshape(TBLP)
  w0p = jnp.pad(
      jnp.transpose(weights_slvra, (1, 0, 2, 3, 4)).reshape(LD, SD * VD, RA),
      ((0, 0), (0, 0), (0, ROWP - RA))).reshape(TBLP)

  gather_call = pl.kernel(
      functools.partial(_bal_gather_body, att, nchunks, niter),
      out_type=[
          jax.ShapeDtypeStruct((b,), jnp.float32),
          jax.ShapeDtypeStruct((b,), jnp.float32),
      ],
      mesh=mesh,
      compiler_params=pltpu.CompilerParams(needs_layout_passes=False),
      scratch_types=(
          [pltpu.VMEM((SLC,), jnp.float32) for _ in range(6)]
          + [pltpu.VMEM((TBLP,), jnp.float32)]
          + [pltpu.VMEM((LN,), jnp.float32), pltpu.VMEM((LN,), jnp.float32)]
          + [pltpu.VMEM((SD * LN,), jnp.float32)]
          + [pltpu.VMEM((CHUNK,), jnp.int32) for _ in range(2)]
          + [pltpu.VMEM((CHUNK,), jnp.float32) for _ in range(4)]
          + [pltpu.VMEM_SHARED((TBLP,), jnp.float32)]
          + [pltpu.VMEM_SHARED((SD * LN,), jnp.float32)]
          + [pltpu.SemaphoreType.DMA for _ in range(4)]
      ),
  )
  bw, bsw = gather_call(hist32, c0p, w0p, source_weights_s, idx)
  return bw, bsw


# speedup vs baseline: 1.0109x; 1.0109x over previous
import functools

import jax
import jax.numpy as jnp
from jax import lax
from jax.experimental import pallas as pl
from jax.experimental.pallas import tpu as pltpu
from jax.experimental.pallas import tpu_sc as plsc

SD, LD, VD, RD, AD = 16, 3, 6, 10, 12
RA = RD * AD
ROWP = 128
L12 = SD * VD * ROWP
TBLP = LD * L12
TSH = VD * ROWP
SLC = LD * TSH
NC, NS, LN = 2, 16, 16
NW = NC * NS
CHUNK = 3200
ATT_PER_DATUM = 0.99999


def _hist_body(nchunks, niter, src_h, lab_h, var_h, ref_h, alt_h,
               idx_h, hist_h,
               b00, b01, b02, b03, b04, b10, b11, b12, b13, b14,
               ix0, ix1, hist_v, sem_in, sem_out):
  wid = lax.axis_index("s") * NC + lax.axis_index("c")
  ins = (src_h, lab_h, var_h, ref_h, alt_h)
  bufs = ((b00, b01, b02, b03, b04), (b10, b11, b12, b13, b14))
  ixv = (ix0, ix1)

  def zero(j, c):
    o = j * (4 * LN)
    for u in range(4):
      hist_v[pl.ds(o + u * LN, LN)] = jnp.zeros((LN,), jnp.float32)
    return c
  lax.fori_loop(0, TBLP // (4 * LN), zero, 0)

  ones = jnp.full((LN,), 1.0, jnp.float32)

  def issue(bsel, cid):
    base = cid * CHUNK
    for h, v in zip(ins, bufs[bsel]):
      pltpu.async_copy(h.at[pl.ds(base, CHUNK)], v, sem_in)

  def drain_in(bsel):
    for h, v in zip(ins, bufs[bsel]):
      pltpu.make_async_copy(h.at[pl.ds(0, CHUNK)], v, sem_in).wait()

  def drain_out(bsel):
    pltpu.make_async_copy(ixv[bsel], idx_h.at[pl.ds(0, CHUNK)], sem_out).wait()

  issue(0, wid)

  def loop_g(g, c):
    for b in range(2):
      cid = wid + (2 * g + b) * NW

      @pl.when(cid < nchunks)
      def _(b=b, cid=cid):
        drain_in(b)

        @pl.when(cid + NW < nchunks)
        def _():
          issue(1 - b, cid + NW)

        @pl.when(g > 0)
        def _():
          drain_out(b)

        sv_v, lv_v, vv_v, rv_v, av_v = bufs[b]
        ixb = ixv[b]

        @plsc.parallel_loop(0, CHUNK, step=LN, unroll=4)
        def _(o):
          s = sv_v[pl.ds(o, LN)]
          l = lv_v[pl.ds(o, LN)]
          v = vv_v[pl.ds(o, LN)]
          r = rv_v[pl.ds(o, LN)]
          a = av_v[pl.ds(o, LN)]
          t = ((l * SD + s) * VD + v) * ROWP + r * AD + a
          ixb[pl.ds(o, LN)] = t | (s << 16)
          plsc.addupdate_scatter(hist_v, [t], ones)
        pltpu.async_copy(ixb, idx_h.at[pl.ds(cid * CHUNK, CHUNK)], sem_out)
    return c
  lax.fori_loop(0, niter // 2, loop_g, 0)
  drain_out(0)
  drain_out(1)

  pltpu.sync_copy(hist_v, hist_h.at[wid])


def _bal_gather_body(att, nchunks, niter, hist_h, c0_h, w0_h, sw0_h, idx_h,
                     bw_h, bsw_h,
                     acc_v, stg0, stg1, c0_v, w0_v, wsl_v, wtab_v, sw16_v,
                     cs_v, ex_v, ix0, ix1, bw0, bw1, bsw0, bsw1,
                     spw_s, spx_s, sem_cfg, sem_tbl, sem_in, sem_out):
  sid = lax.axis_index("s")
  wid = sid * NC + lax.axis_index("c")
  one_m = 1.0 - att
  ixv = (ix0, ix1)
  bwv = (bw0, bw1)
  bswv = (bsw0, bsw1)
  stg = (stg0, stg1)

  pltpu.async_copy(idx_h.at[pl.ds(wid * CHUNK, CHUNK)], ix0, sem_in)

  sbase = sid * TSH
  cfg = []
  for l in range(LD):
    cfg.append(pltpu.async_copy(
        c0_h.at[pl.ds(l * L12 + sbase, TSH)],
        c0_v.at[pl.ds(l * TSH, TSH)], sem_cfg))
    cfg.append(pltpu.async_copy(
        w0_h.at[pl.ds(l * L12 + sbase, TSH)],
        w0_v.at[pl.ds(l * TSH, TSH)], sem_cfg))
  cfg.append(pltpu.async_copy(sw0_h, sw16_v, sem_cfg))

  def issue_w(bsel, w):
    for l in range(LD):
      pltpu.async_copy(
          hist_h.at[w, pl.ds(l * L12 + sbase, TSH)],
          stg[bsel].at[pl.ds(l * TSH, TSH)], sem_tbl)

  def drain_w(bsel):
    for l in range(LD):
      pltpu.make_async_copy(
          hist_h.at[0, pl.ds(l * L12 + sbase, TSH)],
          stg[bsel].at[pl.ds(l * TSH, TSH)], sem_tbl).wait()

  issue_w(0, 0)

  def zacc(j, c):
    o = j * (4 * LN)
    for u in range(4):
      acc_v[pl.ds(o + u * LN, LN)] = jnp.zeros((LN,), jnp.float32)
    return c
  lax.fori_loop(0, SLC // (4 * LN), zacc, 0)

  def wacc(g, c):
    for b in range(2):
      w = 2 * g + b
      drain_w(b)

      @pl.when(w < NW - 1)
      def _(b=b, w=w):
        issue_w(1 - b, w + 1)

      @plsc.parallel_loop(0, SLC, step=LN, unroll=4)
      def _(o, b=b):
        acc_v[pl.ds(o, LN)] = acc_v[pl.ds(o, LN)] + stg[b][pl.ds(o, LN)]
    return c
  lax.fori_loop(0, NW // 2, wacc, 0)

  for d in cfg:
    d.wait()

  def addc(j, c):
    o = j * LN
    acc_v[pl.ds(o, LN)] = acc_v[pl.ds(o, LN)] + c0_v[pl.ds(o, LN)]
    return c
  lax.fori_loop(0, SLC // LN, addc, 0)

  csum = jnp.zeros((LN,), jnp.float32)
  for v in range(VD):
    a_acc = jnp.zeros((LN,), jnp.float32)
    v_acc = jnp.zeros((LN,), jnp.float32)
    u_acc = jnp.zeros((LN,), jnp.float32)
    for j in range(ROWP // LN):
      o = v * ROWP + j * LN
      av = acc_v[pl.ds(o, LN)]
      vv = acc_v[pl.ds(TSH + o, LN)]
      uv = acc_v[pl.ds(2 * TSH + o, LN)]
      a_acc += av
      v_acc += vv
      u_acc += uv
      ratios = (av + 0.01) / (vv + 0.01)
      inv = (vv + 0.01) / (av + 0.01)
      w_art = jnp.clip((1.0 + inv) / 2.0, 0.01, 100.0)
      w_var = jnp.clip((1.0 + ratios) / 2.0, 0.01, 100.0)
      wsl_v[pl.ds(o, LN)] = att * w0_v[pl.ds(o, LN)] + one_m * w_art
      wsl_v[pl.ds(TSH + o, LN)] = (
          att * w0_v[pl.ds(TSH + o, LN)] + one_m * w_var)
    csum += a_acc + v_acc + u_acc
    sa = jnp.broadcast_to(jnp.sum(a_acc), (LN,))
    sv = jnp.broadcast_to(jnp.sum(v_acc), (LN,))
    su = jnp.broadcast_to(jnp.sum(u_acc), (LN,))
    uw = jnp.clip((sa + sv) / su, 0.0, 1.0)
    for j in range(ROWP // LN):
      o = 2 * TSH + v * ROWP + j * LN
      wsl_v[pl.ds(o, LN)] = att * w0_v[pl.ds(o, LN)] + one_m * uw

  cs_v[pl.ds(0, LN)] = jnp.broadcast_to(jnp.sum(csum), (LN,))
  pltpu.sync_copy(cs_v, spx_s.at[pl.ds(sid * LN, LN)])
  for l in range(LD):
    pltpu.sync_copy(wsl_v.at[pl.ds(l * TSH, TSH)],
                    spw_s.at[pl.ds(l * L12 + sbase, TSH)])
  plsc.subcore_barrier()
  pltpu.sync_copy(spw_s, wtab_v)
  pltpu.sync_copy(spx_s, ex_v)

  iota16 = lax.iota(jnp.int32, LN)
  csrow = plsc.load_gather(ex_v, [iota16 * LN])
  totv = jnp.broadcast_to(jnp.sum(csrow), (LN,))
  new_sw = totv / csrow / SD
  sw16_v[pl.ds(0, LN)] = att * sw16_v[pl.ds(0, LN)] + one_m * new_sw

  def drain_gout(bsel):
    pltpu.make_async_copy(bwv[bsel], bw_h.at[pl.ds(0, CHUNK)], sem_out).wait()
    pltpu.make_async_copy(bswv[bsel], bsw_h.at[pl.ds(0, CHUNK)], sem_out).wait()

  def loop_g(g, c):
    for b in range(2):
      cid = wid + (2 * g + b) * NW

      @pl.when(cid < nchunks)
      def _(b=b, cid=cid):
        pltpu.make_async_copy(
            idx_h.at[pl.ds(0, CHUNK)], ixv[b], sem_in).wait()

        @pl.when(cid + NW < nchunks)
        def _():
          pltpu.async_copy(
              idx_h.at[pl.ds((cid + NW) * CHUNK, CHUNK)], ixv[1 - b], sem_in)

        @pl.when(g > 0)
        def _():
          drain_gout(b)

        ixb, bwb, bswb = ixv[b], bwv[b], bswv[b]

        @plsc.parallel_loop(0, CHUNK, step=LN, unroll=4)
        def _(o):
          t = ixb[pl.ds(o, LN)]
          tl = t & 0xFFFF
          s = t >> 16
          bwb[pl.ds(o, LN)] = plsc.load_gather(wtab_v, [tl])
          bswb[pl.ds(o, LN)] = plsc.load_gather(sw16_v, [s])
        base = cid * CHUNK
        pltpu.async_copy(bwb, bw_h.at[pl.ds(base, CHUNK)], sem_out)
        pltpu.async_copy(bswb, bsw_h.at[pl.ds(base, CHUNK)], sem_out)
    return c
  lax.fori_loop(0, niter // 2, loop_g, 0)
  drain_gout(0)
  drain_gout(1)


def kernel(counts_slvra, weights_slvra, source_weights_s,
           sources, labels, var_types, ref_bins, alt_bins):
  b = sources.shape[0]
  assert b % CHUNK == 0
  nchunks = b // CHUNK
  assert nchunks >= 2 * NW
  niter = -(-nchunks // NW)
  niter += niter % 2
  att = float(ATT_PER_DATUM ** b)
  mesh = plsc.VectorSubcoreMesh(core_axis_name="c", subcore_axis_name="s")

  hist_call = pl.kernel(
      functools.partial(_hist_body, nchunks, niter),
      out_type=[
          jax.ShapeDtypeStruct((b,), jnp.int32),
          jax.ShapeDtypeStruct((NW, TBLP), jnp.float32),
      ],
      mesh=mesh,
      compiler_params=pltpu.CompilerParams(needs_layout_passes=False),
      scratch_types=(
          [pltpu.VMEM((CHUNK,), jnp.int32) for _ in range(12)]
          + [pltpu.VMEM((TBLP,), jnp.float32)]
          + [pltpu.SemaphoreType.DMA, pltpu.SemaphoreType.DMA]
      ),
  )
  idx, hist32 = hist_call(sources, labels, var_types, ref_bins, alt_bins)

  c0p = jnp.pad(
      jnp.transpose(counts_slvra, (1, 0, 2, 3, 4)).reshape(LD, SD * VD, RA),
      ((0, 0), (0, 0), (0, ROWP - RA))).reshape(TBLP)
  w0p = jnp.pad(
      jnp.transpose(weights_slvra, (1, 0, 2, 3, 4)).reshape(LD, SD * VD, RA),
      ((0, 0), (0, 0), (0, ROWP - RA))).reshape(TBLP)

  gather_call = pl.kernel(
      functools.partial(_bal_gather_body, att, nchunks, niter),
      out_type=[
          jax.ShapeDtypeStruct((b,), jnp.float32),
          jax.ShapeDtypeStruct((b,), jnp.float32),
      ],
      mesh=mesh,
      compiler_params=pltpu.CompilerParams(needs_layout_passes=False),
      scratch_types=(
          [pltpu.VMEM((SLC,), jnp.float32) for _ in range(6)]
          + [pltpu.VMEM((TBLP,), jnp.float32)]
          + [pltpu.VMEM((LN,), jnp.float32), pltpu.VMEM((LN,), jnp.float32)]
          + [pltpu.VMEM((SD * LN,), jnp.float32)]
          + [pltpu.VMEM((CHUNK,), jnp.int32) for _ in range(2)]
          + [pltpu.VMEM((CHUNK,), jnp.float32) for _ in range(4)]
          + [pltpu.VMEM_SHARED((TBLP,), jnp.float32)]
          + [pltpu.VMEM_SHARED((SD * LN,), jnp.float32)]
          + [pltpu.SemaphoreType.DMA for _ in range(4)]
      ),
  )
  bw, bsw = gather_call(hist32, c0p, w0p, source_weights_s, idx)
  return bw, bsw
